# baseline (speedup 1.0000x reference)
import functools

import jax
import jax.numpy as jnp
from jax import lax
from jax.experimental import pallas as pl
from jax.experimental.pallas import tpu as pltpu
from jax.experimental.pallas import tpu_sc as plsc

N = 10000
D = 128
E = 320000

NC = 2
NS = 16
NW = NC * NS
B = 128
T0 = 100
T1R = 57
K = 2
T1 = 58
CHM = max(T0, T1)
E_PAD = NS * (T0 + T1R) * B
NPAD = 10112
RPT = NPAD // NS


@functools.cache
def _sc_kernels():
    mesh = plsc.VectorSubcoreMesh(core_axis_name="c", subcore_axis_name="s",
                                  num_cores=NC, num_subcores=NS)

    @functools.partial(
        pl.kernel,
        out_type=jax.ShapeDtypeStruct((NC, NPAD, D), jnp.float32),
        mesh=mesh,
        scratch_types=[
            pltpu.VMEM((CHM, 2, B), jnp.int32),
            pltpu.VMEM((B, D), jnp.float32),
            pltpu.VMEM_SHARED((NPAD, D), jnp.float32),
            pltpu.SemaphoreType.DMA,
        ],
    )
    def deg_kernel(ed_hbm, ones_hbm, zeros_hbm, out_hbm,
                   ed_v, ones_v, acc, sem_s):
        c = lax.axis_index("c")
        s = lax.axis_index("s")
        wid = s * NC + c
        trip = jnp.where(c == 0, T0, T1)
        pltpu.sync_copy(zeros_hbm, acc.at[pl.ds(s * RPT, RPT)])
        pltpu.sync_copy(ones_hbm, ones_v)
        pltpu.sync_copy(ed_hbm.at[wid], ed_v)
        plsc.subcore_barrier()

        def fire(j, carry):
            pltpu.async_copy(ones_v, acc.at[ed_v.at[j, 1]], sem_s, add=True)
            return carry

        def drain(j, carry):
            pltpu.make_async_copy(ones_v, acc.at[ed_v.at[0, 1]],
                                  sem_s).wait()
            return carry

        lax.fori_loop(0, trip, fire, 0)
        lax.fori_loop(0, trip, drain, 0)
        plsc.subcore_barrier()
        pltpu.sync_copy(acc.at[pl.ds(s * RPT, RPT)],
                        out_hbm.at[c, pl.ds(s * RPT, RPT)])

    @functools.partial(
        pl.kernel,
        out_type=jax.ShapeDtypeStruct((NC, NPAD, D), jnp.float32),
        mesh=mesh,
        scratch_types=[
            pltpu.VMEM((CHM, B), jnp.int32),
            pltpu.VMEM((K, 2, B), jnp.int32),
            [pltpu.VMEM((B, D), jnp.float32) for _ in range(K)],
            pltpu.VMEM_SHARED((NPAD, D), jnp.float32),
            pltpu.SemaphoreType.DMA,
            pltpu.SemaphoreType.DMA,
        ],
    )
    def agg_kernel(edp_hbm, hs_hbm, zeros_hbm, out_hbm,
                   edp_v, idx_u, rows, acc, sem_g, sem_s):
        c = lax.axis_index("c")
        s = lax.axis_index("s")
        wid = s * NC + c
        trip = jnp.where(c == 0, T0, T1)
        pltpu.sync_copy(zeros_hbm, acc.at[pl.ds(s * RPT, RPT)])
        pltpu.sync_copy(edp_hbm.at[wid], edp_v)
        plsc.subcore_barrier()

        def phase(p, carry):
            for k in range(K):
                j = p * K + k
                for q in range(B // 16):
                    pk = edp_v[j, pl.ds(q * 16, 16)]
                    idx_u[k, 0, pl.ds(q * 16, 16)] = lax.shift_right_logical(
                        pk, 16)
                    idx_u[k, 1, pl.ds(q * 16, 16)] = lax.bitwise_and(
                        pk, 0xFFFF)
            for k in range(K):
                pltpu.async_copy(hs_hbm.at[idx_u.at[k, 0]], rows[k], sem_g)
            for k in range(K):
                pltpu.make_async_copy(hs_hbm.at[idx_u.at[0, 0]],
                                      rows[k], sem_g).wait()
            for k in range(K):
                pltpu.async_copy(rows[k], acc.at[idx_u.at[k, 1]],
                                 sem_s, add=True)
            for k in range(K):
                pltpu.make_async_copy(rows[0], acc.at[idx_u.at[0, 1]],
                                      sem_s).wait()
            return carry

        lax.fori_loop(0, trip // K, phase, 0)
        plsc.subcore_barrier()
        pltpu.sync_copy(acc.at[pl.ds(s * RPT, RPT)],
                        out_hbm.at[c, pl.ds(s * RPT, RPT)])

    return deg_kernel, agg_kernel


def _tc_stage1(x, W1, degP):
    def body(x_ref, w_ref, degp_ref, dinv_ref, h_ref, hs_ref):
        deg = degp_ref[0, :N, 0:1] + degp_ref[1, :N, 0:1] + 1.0
        dinv = lax.rsqrt(deg)
        h = jnp.dot(x_ref[...], w_ref[...], preferred_element_type=jnp.float32)
        dinv_ref[...] = dinv
        h_ref[...] = h
        hs_ref[...] = h * dinv

    return pl.pallas_call(
        body,
        out_shape=[
            jax.ShapeDtypeStruct((N, 1), jnp.float32),
            jax.ShapeDtypeStruct((N, D), jnp.float32),
            jax.ShapeDtypeStruct((N, D), jnp.float32),
        ],
    )(x, W1, degP)


def _tc_mid(S, h, dinv, b, gamma, beta, W_next):
    def body(s_ref, h_ref, dinv_ref, b_ref, g_ref, be_ref, w_ref,
             h2_ref, hs2_ref):
        dinv = dinv_ref[...]
        h = h_ref[...]
        agg = s_ref[0, :N, :] + s_ref[1, :N, :]
        conv = dinv * agg + (dinv * dinv) * h + b_ref[...]
        mean = jnp.mean(conv, axis=0, keepdims=True)
        var = jnp.mean((conv - mean) ** 2, axis=0, keepdims=True)
        y = g_ref[...] * (conv - mean) * lax.rsqrt(var + 1e-5) + be_ref[...]
        y = jnp.maximum(y, 0.0)
        h2 = jnp.dot(y, w_ref[...], preferred_element_type=jnp.float32)
        h2_ref[...] = h2
        hs2_ref[...] = h2 * dinv

    return pl.pallas_call(
        body,
        out_shape=[
            jax.ShapeDtypeStruct((N, D), jnp.float32),
            jax.ShapeDtypeStruct((N, D), jnp.float32),
        ],
    )(S, h, dinv, b.reshape(1, D), gamma.reshape(1, D), beta.reshape(1, D),
      W_next)


def _tc_final(S, h, dinv, b):
    def body(s_ref, h_ref, dinv_ref, b_ref, out_ref):
        dinv = dinv_ref[...]
        agg = s_ref[0, :N, :] + s_ref[1, :N, :]
        conv = dinv * agg + (dinv * dinv) * h_ref[...] + b_ref[...]
        m = jnp.max(conv, axis=-1, keepdims=True)
        z = conv - m
        lse = jnp.log(jnp.sum(jnp.exp(z), axis=-1, keepdims=True))
        out_ref[...] = z - lse

    return pl.pallas_call(
        body,
        out_shape=jax.ShapeDtypeStruct((N, D), jnp.float32),
    )(S, h, dinv, b.reshape(1, D))


def kernel(x, edge_index, W1, b1, W2, b2, W3, b3, gamma1, beta1,
           gamma2, beta2):
    src = edge_index[0].astype(jnp.int32)
    dst = edge_index[1].astype(jnp.int32)
    pad = E_PAD - E
    srcp = jnp.concatenate([src, jnp.zeros((pad,), jnp.int32)]
                           ).reshape(-1, 1, B)
    dstp = jnp.concatenate([dst, jnp.full((pad,), N, jnp.int32)]
                           ).reshape(-1, 1, B)
    edf = jnp.concatenate([srcp, dstp], axis=1)
    ed0 = edf[:NS * T0].reshape(NS, 1, T0, 2, B)
    ed1 = edf[NS * T0:].reshape(NS, 1, T1R, 2, B)
    fill = jnp.concatenate(
        [jnp.zeros((NS, 1, CHM - T1R, 1, B), jnp.int32),
         jnp.full((NS, 1, CHM - T1R, 1, B), N, jnp.int32)], axis=3)
    ed1 = jnp.concatenate([ed1, fill], axis=2)
    ed = jnp.concatenate([ed0, ed1], axis=1).reshape(NW, CHM, 2, B)
    edp = jnp.bitwise_or(jnp.left_shift(ed[:, :, 0], 16), ed[:, :, 1])

    ones = jnp.ones((B, D), jnp.float32)
    zeros = jnp.zeros((RPT, D), jnp.float32)

    deg_kernel, agg_kernel = _sc_kernels()
    degP = deg_kernel(ed, ones, zeros)
    dinv, h1, hs1 = _tc_stage1(x, W1, degP)
    S1 = agg_kernel(edp, hs1, zeros)
    h2, hs2 = _tc_mid(S1, h1, dinv, b1, gamma1, beta1, W2)
    S2 = agg_kernel(edp, hs2, zeros)
    h3, hs3 = _tc_mid(S2, h2, dinv, b2, gamma2, beta2, W3)
    S3 = agg_kernel(edp, hs3, zeros)
    return _tc_final(S3, h3, dinv, b3)


# speedup vs baseline: 1.1598x; 1.1598x over previous
import functools

import jax
import jax.numpy as jnp
from jax import lax
from jax.experimental import pallas as pl
from jax.experimental.pallas import tpu as pltpu
from jax.experimental.pallas import tpu_sc as plsc

N = 10000
D = 128
E = 320000

NC = 2
NS = 16
NW = NC * NS
B = 128
T0 = 100
T1R = 57
T1 = T1R
CHM = max(T0, T1)
E_PAD = NS * (T0 + T1R) * B
NPAD = 10112
RPT = NPAD // NS


@functools.cache
def _sc_kernels():
    mesh = plsc.VectorSubcoreMesh(core_axis_name="c", subcore_axis_name="s",
                                  num_cores=NC, num_subcores=NS)

    @functools.partial(
        pl.kernel,
        out_type=jax.ShapeDtypeStruct((NC, NPAD, D), jnp.float32),
        mesh=mesh,
        scratch_types=[
            pltpu.VMEM((CHM, 2, B), jnp.int32),
            pltpu.VMEM((B, D), jnp.float32),
            pltpu.VMEM_SHARED((NPAD, D), jnp.float32),
            pltpu.SemaphoreType.DMA,
        ],
    )
    def deg_kernel(ed_hbm, ones_hbm, zeros_hbm, out_hbm,
                   ed_v, ones_v, acc, sem_s):
        c = lax.axis_index("c")
        s = lax.axis_index("s")
        wid = s * NC + c
        trip = jnp.where(c == 0, T0, T1)
        pltpu.sync_copy(zeros_hbm, acc.at[pl.ds(s * RPT, RPT)])
        pltpu.sync_copy(ones_hbm, ones_v)
        pltpu.sync_copy(ed_hbm.at[wid], ed_v)
        plsc.subcore_barrier()

        def fire(j, carry):
            pltpu.async_copy(ones_v, acc.at[ed_v.at[j, 1]], sem_s, add=True)
            return carry

        def drain(j, carry):
            pltpu.make_async_copy(ones_v, acc.at[ed_v.at[0, 1]],
                                  sem_s).wait()
            return carry

        lax.fori_loop(0, trip, fire, 0)
        lax.fori_loop(0, trip, drain, 0)
        plsc.subcore_barrier()
        pltpu.sync_copy(acc.at[pl.ds(s * RPT, RPT)],
                        out_hbm.at[c, pl.ds(s * RPT, RPT)])

    @functools.partial(
        pl.kernel,
        out_type=jax.ShapeDtypeStruct((NC, NPAD, D), jnp.float32),
        mesh=mesh,
        scratch_types=[
            pltpu.VMEM((CHM, 2, B), jnp.int32),
            pltpu.VMEM((B, D), jnp.float32),
            pltpu.VMEM_SHARED((NPAD, D), jnp.float32),
            pltpu.SemaphoreType.DMA,
        ],
    )
    def agg_kernel(ed_hbm, hs_hbm, zeros_hbm, out_hbm,
                   ed_v, rows_v, acc, sem):
        c = lax.axis_index("c")
        s = lax.axis_index("s")
        wid = s * NC + c
        trip = jnp.where(c == 0, T0, T1)
        pltpu.sync_copy(zeros_hbm, acc.at[pl.ds(s * RPT, RPT)])
        pltpu.sync_copy(ed_hbm.at[wid], ed_v)
        plsc.subcore_barrier()

        def body(j, carry):
            pltpu.async_copy(hs_hbm.at[ed_v.at[j, 0]], rows_v, sem).wait()
            pltpu.sync_copy(rows_v, acc.at[ed_v.at[j, 1]], add=True)
            return carry

        lax.fori_loop(0, trip, body, 0)
        plsc.subcore_barrier()
        pltpu.sync_copy(acc.at[pl.ds(s * RPT, RPT)],
                        out_hbm.at[c, pl.ds(s * RPT, RPT)])

    return deg_kernel, agg_kernel


def _tc_stage1(x, W1, degP):
    def body(x_ref, w_ref, degp_ref, dinv_ref, h_ref, hs_ref):
        deg = degp_ref[0, :N, 0:1] + degp_ref[1, :N, 0:1] + 1.0
        dinv = lax.rsqrt(deg)
        h = jnp.dot(x_ref[...], w_ref[...], preferred_element_type=jnp.float32)
        dinv_ref[...] = dinv
        h_ref[...] = h
        hs_ref[...] = h * dinv

    return pl.pallas_call(
        body,
        out_shape=[
            jax.ShapeDtypeStruct((N, 1), jnp.float32),
            jax.ShapeDtypeStruct((N, D), jnp.float32),
            jax.ShapeDtypeStruct((N, D), jnp.float32),
        ],
    )(x, W1, degP)


def _tc_mid(S, h, dinv, b, gamma, beta, W_next):
    def body(s_ref, h_ref, dinv_ref, b_ref, g_ref, be_ref, w_ref,
             h2_ref, hs2_ref):
        dinv = dinv_ref[...]
        h = h_ref[...]
        agg = s_ref[0, :N, :] + s_ref[1, :N, :]
        conv = dinv * agg + (dinv * dinv) * h + b_ref[...]
        mean = jnp.mean(conv, axis=0, keepdims=True)
        var = jnp.mean((conv - mean) ** 2, axis=0, keepdims=True)
        y = g_ref[...] * (conv - mean) * lax.rsqrt(var + 1e-5) + be_ref[...]
        y = jnp.maximum(y, 0.0)
        h2 = jnp.dot(y, w_ref[...], preferred_element_type=jnp.float32)
        h2_ref[...] = h2
        hs2_ref[...] = h2 * dinv

    return pl.pallas_call(
        body,
        out_shape=[
            jax.ShapeDtypeStruct((N, D), jnp.float32),
            jax.ShapeDtypeStruct((N, D), jnp.float32),
        ],
    )(S, h, dinv, b.reshape(1, D), gamma.reshape(1, D), beta.reshape(1, D),
      W_next)


def _tc_final(S, h, dinv, b):
    def body(s_ref, h_ref, dinv_ref, b_ref, out_ref):
        dinv = dinv_ref[...]
        agg = s_ref[0, :N, :] + s_ref[1, :N, :]
        conv = dinv * agg + (dinv * dinv) * h_ref[...] + b_ref[...]
        m = jnp.max(conv, axis=-1, keepdims=True)
        z = conv - m
        lse = jnp.log(jnp.sum(jnp.exp(z), axis=-1, keepdims=True))
        out_ref[...] = z - lse

    return pl.pallas_call(
        body,
        out_shape=jax.ShapeDtypeStruct((N, D), jnp.float32),
    )(S, h, dinv, b.reshape(1, D))


def kernel(x, edge_index, W1, b1, W2, b2, W3, b3, gamma1, beta1,
           gamma2, beta2):
    src = edge_index[0].astype(jnp.int32)
    dst = edge_index[1].astype(jnp.int32)
    pad = E_PAD - E
    srcp = jnp.concatenate([src, jnp.zeros((pad,), jnp.int32)]
                           ).reshape(-1, 1, B)
    dstp = jnp.concatenate([dst, jnp.full((pad,), N, jnp.int32)]
                           ).reshape(-1, 1, B)
    edf = jnp.concatenate([srcp, dstp], axis=1)
    ed0 = edf[:NS * T0].reshape(NS, 1, T0, 2, B)
    ed1 = edf[NS * T0:].reshape(NS, 1, T1R, 2, B)
    fill = jnp.concatenate(
        [jnp.zeros((NS, 1, CHM - T1R, 1, B), jnp.int32),
         jnp.full((NS, 1, CHM - T1R, 1, B), N, jnp.int32)], axis=3)
    ed1 = jnp.concatenate([ed1, fill], axis=2)
    ed = jnp.concatenate([ed0, ed1], axis=1).reshape(NW, CHM, 2, B)

    ones = jnp.ones((B, D), jnp.float32)
    zeros = jnp.zeros((RPT, D), jnp.float32)

    deg_kernel, agg_kernel = _sc_kernels()
    degP = deg_kernel(ed, ones, zeros)
    dinv, h1, hs1 = _tc_stage1(x, W1, degP)
    S1 = agg_kernel(ed, hs1, zeros)
    h2, hs2 = _tc_mid(S1, h1, dinv, b1, gamma1, beta1, W2)
    S2 = agg_kernel(ed, hs2, zeros)
    h3, hs3 = _tc_mid(S2, h2, dinv, b2, gamma2, beta2, W3)
    S3 = agg_kernel(ed, hs3, zeros)
    return _tc_final(S3, h3, dinv, b3)
